# baseline (speedup 1.0000x reference)
import jax
import jax.numpy as jnp
from jax.experimental import pallas as pl
from jax.experimental.pallas import tpu as pltpu

_D = 128
_C = 8192
_NB = 8


def _tc_copy_kernel(rows):
    n_per = rows // _C

    def body(k_in, v_in, k_out, v_out, *scratch):
        bufs = scratch[:_NB]
        sin = scratch[_NB : 2 * _NB]
        sout = scratch[2 * _NB :]
        n = 2 * n_per

        def src_dst_off(i):
            if i < n_per:
                return k_in, k_out, i * _C
            return v_in, v_out, (i - n_per) * _C

        in_copies = [None] * n
        out_copies = [None] * n
        for i in range(min(_NB, n)):
            src, _, off = src_dst_off(i)
            in_copies[i] = pltpu.make_async_copy(
                src.at[pl.ds(off, _C)], bufs[i % _NB], sin[i % _NB]
            )
            in_copies[i].start()
        for i in range(n):
            b = i % _NB
            if i >= _NB:
                out_copies[i - _NB].wait()
                src, _, off = src_dst_off(i)
                in_copies[i] = pltpu.make_async_copy(
                    src.at[pl.ds(off, _C)], bufs[b], sin[b]
                )
                in_copies[i].start()
            in_copies[i].wait()
            _, dst, off = src_dst_off(i)
            out_copies[i] = pltpu.make_async_copy(
                bufs[b], dst.at[pl.ds(off, _C)], sout[b]
            )
            out_copies[i].start()
        for i in range(max(0, n - _NB), n):
            out_copies[i].wait()

    return pl.pallas_call(
        body,
        out_shape=(
            jax.ShapeDtypeStruct((rows, _D), jnp.float32),
            jax.ShapeDtypeStruct((rows, _D), jnp.float32),
        ),
        in_specs=[
            pl.BlockSpec(memory_space=pl.ANY),
            pl.BlockSpec(memory_space=pl.ANY),
        ],
        out_specs=[
            pl.BlockSpec(memory_space=pl.ANY),
            pl.BlockSpec(memory_space=pl.ANY),
        ],
        scratch_shapes=(
            [pltpu.VMEM((_C, _D), jnp.float32) for _ in range(_NB)]
            + [pltpu.SemaphoreType.DMA for _ in range(2 * _NB)]
        ),
    )


def kernel(key_states, value_states, layer_idx):
    shape = key_states.shape
    rows = shape[0] * shape[1] * shape[2]
    k2 = key_states.reshape(rows, _D)
    v2 = value_states.reshape(rows, _D)
    ko, vo = _tc_copy_kernel(rows)(k2, v2)
    return (ko.reshape(shape), vo.reshape(shape))


# speedup vs baseline: 1.2691x; 1.2691x over previous
import functools

import jax
import jax.numpy as jnp
from jax import lax
from jax.experimental import pallas as pl
from jax.experimental.pallas import tpu as pltpu
from jax.experimental.pallas import tpu_sc as plsc

_D = 128
_CS = 512
_NB = 2


def _sc_copy_one(rows):
    info = plsc.get_sparse_core_info()
    nc, ns = info.num_cores, info.num_subcores
    nw = nc * ns
    rpw = rows // nw
    n = rpw // _CS

    mesh = plsc.VectorSubcoreMesh(core_axis_name="c", subcore_axis_name="s")

    @functools.partial(
        pl.kernel,
        mesh=mesh,
        out_type=jax.ShapeDtypeStruct((rows, _D), jnp.float32),
        scratch_types=(
            [pltpu.VMEM_SHARED((ns, _CS, _D), jnp.float32) for _ in range(_NB)]
            + [pltpu.SemaphoreType.DMA for _ in range(2 * _NB)]
        ),
    )
    def sc_copy(src_hbm, dst_hbm, *scratch):
        shared = scratch[:_NB]
        sin = scratch[_NB : _NB + _NB]
        sout = scratch[2 * _NB :]
        cid = lax.axis_index("c")
        sid = lax.axis_index("s")
        wid = sid * nc + cid
        base = wid * rpw
        bufs = [shared[b].at[sid] for b in range(_NB)]

        in_copies = [None] * n
        out_copies = [None] * n
        for i in range(min(_NB, n)):
            in_copies[i] = pltpu.async_copy(
                src_hbm.at[pl.ds(base + i * _CS, _CS)], bufs[i % _NB], sin[i % _NB]
            )
        for i in range(n):
            b = i % _NB
            if i >= _NB:
                out_copies[i - _NB].wait()
                in_copies[i] = pltpu.async_copy(
                    src_hbm.at[pl.ds(base + i * _CS, _CS)], bufs[b], sin[b]
                )
            in_copies[i].wait()
            out_copies[i] = pltpu.async_copy(
                bufs[b], dst_hbm.at[pl.ds(base + i * _CS, _CS)], sout[b]
            )
        for i in range(max(0, n - _NB), n):
            out_copies[i].wait()

    return sc_copy


def _tc_copy_body(in_ref, out_ref):
    out_ref[...] = in_ref[...]


def _tc_copy_one(rows):
    blk = 8192
    spec = pl.BlockSpec((blk, _D), lambda i: (i, 0))
    return pl.pallas_call(
        _tc_copy_body,
        grid=(rows // blk,),
        out_shape=jax.ShapeDtypeStruct((rows, _D), jnp.float32),
        in_specs=[spec],
        out_specs=spec,
    )


def kernel(key_states, value_states, layer_idx):
    shape = key_states.shape
    rows = shape[0] * shape[1] * shape[2]
    k2 = key_states.reshape(rows, _D)
    v2 = value_states.reshape(rows, _D)
    ko = _tc_copy_one(rows)(k2)
    vo = _sc_copy_one(rows)(v2)
    return (ko.reshape(shape), vo.reshape(shape))
